# baseline (speedup 1.0000x reference)
import jax
import jax.numpy as jnp
from jax import lax
from jax.experimental import pallas as pl
from jax.experimental.pallas import tpu as pltpu
from jax.experimental.pallas import tpu_sc as plsc

N_FIELDS = 26
FIELD_SIZE = 100000
D = 32
DP = 128
B = 16384
V = 2600000
L = 16
NC, NS = 2, 16
NW = NC * NS
BPW = B // NW
IPW = BPW * N_FIELDS
CHUNK = 208
SUB = 104
GB = CHUNK // N_FIELDS
NCH = IPW // CHUNK
PERIOD = 208
INV_N = float(1.0 / N_FIELDS)

DT_Q = 2048
DT_W = 4 * DT_Q
DT_GRID = -(-V // DT_W)
VP = DT_GRID * DT_Q


def _detile_body(in_ref, out_ref, scratch, sem):
    i = pl.program_id(0)
    p = i % 2

    @pl.when(i >= 2)
    def _():
        pltpu.make_async_copy(
            scratch.at[p],
            out_ref.at[pl.ds((i - 2) * DT_Q, DT_Q)], sem.at[p]).wait()

    eye = jnp.eye(D, dtype=jnp.float32)
    for c in range(4):
        xc = in_ref[:, pl.ds(c * DT_Q, DT_Q)]
        xt = lax.dot_general(xc, eye, (((0,), (0,)), ((), ())),
                             precision=lax.Precision.HIGHEST)
        scratch[p, :, pl.ds(c * D, D)] = xt

    pltpu.async_copy(
        scratch.at[p], out_ref.at[pl.ds(i * DT_Q, DT_Q)], sem.at[p])

    @pl.when(i == DT_GRID - 1)
    def _():
        pltpu.make_async_copy(
            scratch.at[1 - p],
            out_ref.at[pl.ds(0, DT_Q)], sem.at[1 - p]).wait()
        pltpu.make_async_copy(
            scratch.at[p],
            out_ref.at[pl.ds(0, DT_Q)], sem.at[p]).wait()


def _tc_detile(tt):
    return pl.pallas_call(
        _detile_body,
        grid=(DT_GRID,),
        in_specs=[pl.BlockSpec((D, DT_W), lambda i: (0, i))],
        out_specs=pl.BlockSpec(memory_space=pl.ANY),
        out_shape=jax.ShapeDtypeStruct((VP, DP), jnp.float32),
        scratch_shapes=[
            pltpu.VMEM((2, DT_Q, DP), jnp.float32),
            pltpu.SemaphoreType.DMA((2,)),
        ],
    )(tt)


def _fire(table_hbm, idx_v, rows_v, sem, c, b):
    for s in range(CHUNK // SUB):
        src = table_hbm.at[idx_v.at[pl.ds(c * CHUNK + s * SUB, SUB)]]
        pltpu.async_copy(src, rows_v.at[b, pl.ds(s * SUB, SUB)], sem.at[b])


def _drain(table_hbm, idx_v, rows_v, sem, b):
    for s in range(CHUNK // SUB):
        pltpu.make_async_copy(
            table_hbm.at[idx_v.at[pl.ds(s * SUB, SUB)]],
            rows_v.at[b, pl.ds(s * SUB, SUB)], sem.at[b]).wait()


def _sc_body(idx_hbm, table_hbm, out_hbm, idx_v, lb_v, rows_v, out_v, sem):
    wid = lax.axis_index("s") * NC + lax.axis_index("c")

    pltpu.sync_copy(idx_hbm.at[pl.ds(wid * IPW, IPW)], idx_v)

    def off_body(blk, carry):
        base = blk * PERIOD
        for v in range(PERIOD // L):
            off = ((lax.iota(jnp.int32, L) + v * L) % N_FIELDS) * FIELD_SIZE
            sl = pl.ds(base + v * L, L)
            r = idx_v[sl] + off
            idx_v[sl] = ((r >> 13) << 11) | (r & 2047)
            lb_v[sl] = ((r >> 11) & 3) << 5
        return carry

    lax.fori_loop(0, IPW // PERIOD, off_body, 0)

    for b in range(2):
        _fire(table_hbm, idx_v, rows_v, sem, b, b)

    def ring_body(i, carry):
        for b in range(2):
            c = 2 * i + b
            _drain(table_hbm, idx_v, rows_v, sem, b)
            lbase = c * CHUNK
            lbv = [lb_v[pl.ds(lbase + v * L, L)] for v in range(CHUNK // L)]
            for ii in range(GB):
                r0 = ii * N_FIELDS
                cb = lbv[r0 // L][r0 % L]
                acc0 = rows_v[b, r0, pl.ds(cb, L)]
                acc1 = rows_v[b, r0, pl.ds(cb + L, L)]
                for j in range(1, N_FIELDS):
                    k = r0 + j
                    cj = lbv[k // L][k % L]
                    acc0 = acc0 + rows_v[b, k, pl.ds(cj, L)]
                    acc1 = acc1 + rows_v[b, k, pl.ds(cj + L, L)]
                orow = c * GB + ii
                out_v[orow, pl.ds(0, L)] = acc0 * INV_N
                out_v[orow, pl.ds(L, L)] = acc1 * INV_N

            @pl.when(c + 2 < NCH)
            def _():
                _fire(table_hbm, idx_v, rows_v, sem, c + 2, b)
        return carry

    lax.fori_loop(0, NCH // 2, ring_body, 0)

    pltpu.sync_copy(out_v, out_hbm.at[pl.ds(wid * BPW, BPW)])


@jax.jit
def _sc_embed(idx_flat, table):
    mesh = plsc.VectorSubcoreMesh(core_axis_name="c", subcore_axis_name="s")
    return pl.kernel(
        _sc_body,
        out_type=jax.ShapeDtypeStruct((B, D), jnp.float32),
        mesh=mesh,
        scratch_types=[
            pltpu.VMEM((IPW,), jnp.int32),
            pltpu.VMEM((IPW,), jnp.int32),
            pltpu.VMEM((2, CHUNK, DP), jnp.float32),
            pltpu.VMEM((BPW, D), jnp.float32),
            pltpu.SemaphoreType.DMA((2,)),
        ],
        compiler_params=pltpu.CompilerParams(use_tc_tiling_on_sc=False),
    )(idx_flat, table)


def kernel(inputs, embedding):
    table_pack = _tc_detile(embedding.T)
    return _sc_embed(inputs.reshape(-1), table_pack)


# speedup vs baseline: 1.9117x; 1.9117x over previous
import jax
import jax.numpy as jnp
from jax import lax
from jax.experimental import pallas as pl
from jax.experimental.pallas import tpu as pltpu
from jax.experimental.pallas import tpu_sc as plsc

N_FIELDS = 26
FIELD_SIZE = 100000
D = 32
DP = 128
B = 16384
V = 2600000
L = 16
NC, NS = 2, 16
NW = NC * NS
BPW = B // NW
IPW = BPW * N_FIELDS
CHUNK = 208
SUB = 104
GB = CHUNK // N_FIELDS
NCH = IPW // CHUNK
PERIOD = 208
INV_N = float(1.0 / N_FIELDS)

DT_Q = 2048
DT_W = 4 * DT_Q
DT_GRID = -(-V // DT_W)
VP = DT_GRID * DT_Q


def _detile_body(in_ref, out_ref, scratch, sem):
    i = pl.program_id(0)
    p = i % 2

    @pl.when(i >= 2)
    def _():
        pltpu.make_async_copy(
            scratch.at[p],
            out_ref.at[pl.ds((i - 2) * DT_Q, DT_Q)], sem.at[p]).wait()

    for c in range(4):
        xc = in_ref[:, pl.ds(c * DT_Q, DT_Q)]
        scratch[p, :, pl.ds(c * D, D)] = xc.T

    pltpu.async_copy(
        scratch.at[p], out_ref.at[pl.ds(i * DT_Q, DT_Q)], sem.at[p])

    @pl.when(i == DT_GRID - 1)
    def _():
        pltpu.make_async_copy(
            scratch.at[1 - p],
            out_ref.at[pl.ds(0, DT_Q)], sem.at[1 - p]).wait()
        pltpu.make_async_copy(
            scratch.at[p],
            out_ref.at[pl.ds(0, DT_Q)], sem.at[p]).wait()


def _tc_detile(tt):
    return pl.pallas_call(
        _detile_body,
        grid=(DT_GRID,),
        in_specs=[pl.BlockSpec((D, DT_W), lambda i: (0, i))],
        out_specs=pl.BlockSpec(memory_space=pl.ANY),
        out_shape=jax.ShapeDtypeStruct((VP, DP), jnp.float32),
        scratch_shapes=[
            pltpu.VMEM((2, DT_Q, DP), jnp.float32),
            pltpu.SemaphoreType.DMA((2,)),
        ],
    )(tt)


def _fire(table_hbm, idx_v, rows_v, sem, c, b):
    for s in range(CHUNK // SUB):
        src = table_hbm.at[idx_v.at[pl.ds(c * CHUNK + s * SUB, SUB)]]
        pltpu.async_copy(src, rows_v.at[b, pl.ds(s * SUB, SUB)], sem.at[b])


def _drain(table_hbm, idx_v, rows_v, sem, b):
    for s in range(CHUNK // SUB):
        pltpu.make_async_copy(
            table_hbm.at[idx_v.at[pl.ds(s * SUB, SUB)]],
            rows_v.at[b, pl.ds(s * SUB, SUB)], sem.at[b]).wait()


def _sc_body(idx_hbm, table_hbm, out_hbm, idx_v, lb_v, rows_v, out_v, sem):
    wid = lax.axis_index("s") * NC + lax.axis_index("c")

    pltpu.sync_copy(idx_hbm.at[pl.ds(wid * IPW, IPW)], idx_v)

    def off_body(blk, carry):
        base = blk * PERIOD
        for v in range(PERIOD // L):
            off = ((lax.iota(jnp.int32, L) + v * L) % N_FIELDS) * FIELD_SIZE
            sl = pl.ds(base + v * L, L)
            r = idx_v[sl] + off
            idx_v[sl] = ((r >> 13) << 11) | (r & 2047)
            lb_v[sl] = ((r >> 11) & 3) << 5
        return carry

    lax.fori_loop(0, IPW // PERIOD, off_body, 0)

    for b in range(2):
        _fire(table_hbm, idx_v, rows_v, sem, b, b)

    def ring_body(i, carry):
        for b in range(2):
            c = 2 * i + b
            _drain(table_hbm, idx_v, rows_v, sem, b)
            lbase = c * CHUNK
            lbv = [lb_v[pl.ds(lbase + v * L, L)] for v in range(CHUNK // L)]
            for ii in range(GB):
                r0 = ii * N_FIELDS
                cb = lbv[r0 // L][r0 % L]
                acc0 = rows_v[b, r0, pl.ds(cb, L)]
                acc1 = rows_v[b, r0, pl.ds(cb + L, L)]
                for j in range(1, N_FIELDS):
                    k = r0 + j
                    cj = lbv[k // L][k % L]
                    acc0 = acc0 + rows_v[b, k, pl.ds(cj, L)]
                    acc1 = acc1 + rows_v[b, k, pl.ds(cj + L, L)]
                orow = c * GB + ii
                out_v[orow, pl.ds(0, L)] = acc0 * INV_N
                out_v[orow, pl.ds(L, L)] = acc1 * INV_N

            @pl.when(c + 2 < NCH)
            def _():
                _fire(table_hbm, idx_v, rows_v, sem, c + 2, b)
        return carry

    lax.fori_loop(0, NCH // 2, ring_body, 0)

    pltpu.sync_copy(out_v, out_hbm.at[pl.ds(wid * BPW, BPW)])


@jax.jit
def _sc_embed(idx_flat, table):
    mesh = plsc.VectorSubcoreMesh(core_axis_name="c", subcore_axis_name="s")
    return pl.kernel(
        _sc_body,
        out_type=jax.ShapeDtypeStruct((B, D), jnp.float32),
        mesh=mesh,
        scratch_types=[
            pltpu.VMEM((IPW,), jnp.int32),
            pltpu.VMEM((IPW,), jnp.int32),
            pltpu.VMEM((2, CHUNK, DP), jnp.float32),
            pltpu.VMEM((BPW, D), jnp.float32),
            pltpu.SemaphoreType.DMA((2,)),
        ],
        compiler_params=pltpu.CompilerParams(use_tc_tiling_on_sc=False),
    )(idx_flat, table)


def kernel(inputs, embedding):
    table_pack = _tc_detile(embedding.T)
    return _sc_embed(inputs.reshape(-1), table_pack)


# speedup vs baseline: 1.9518x; 1.0210x over previous
import jax
import jax.numpy as jnp
from jax import lax
from jax.experimental import pallas as pl
from jax.experimental.pallas import tpu as pltpu
from jax.experimental.pallas import tpu_sc as plsc

N_FIELDS = 26
FIELD_SIZE = 100000
D = 32
DP = 128
B = 16384
V = 2600000
L = 16
NC, NS = 2, 16
NW = NC * NS
BPW = B // NW
IPW = BPW * N_FIELDS
CHUNK = 208
SUB = 104
GB = CHUNK // N_FIELDS
NCH = IPW // CHUNK
PERIOD = 208
INV_N = float(1.0 / N_FIELDS)

DT_Q = 2048
DT_S = 2
DT_W = 4 * DT_Q * DT_S
DT_GRID = -(-V // DT_W)
VP = DT_GRID * DT_Q * DT_S


def _detile_body(in_ref, out_ref, scratch, sem):
    i = pl.program_id(0)
    p = i % 2

    QQ = DT_Q * DT_S

    @pl.when(i >= 2)
    def _():
        pltpu.make_async_copy(
            scratch.at[p],
            out_ref.at[pl.ds((i - 2) * QQ, QQ)], sem.at[p]).wait()

    for c2 in range(4 * DT_S):
        q, c = divmod(c2, 4)
        xc = in_ref[:, pl.ds(c2 * DT_Q, DT_Q)]
        scratch[p, pl.ds(q * DT_Q, DT_Q), pl.ds(c * D, D)] = xc.T

    pltpu.async_copy(
        scratch.at[p], out_ref.at[pl.ds(i * QQ, QQ)], sem.at[p])

    @pl.when(i == DT_GRID - 1)
    def _():
        pltpu.make_async_copy(
            scratch.at[1 - p],
            out_ref.at[pl.ds(0, QQ)], sem.at[1 - p]).wait()
        pltpu.make_async_copy(
            scratch.at[p],
            out_ref.at[pl.ds(0, QQ)], sem.at[p]).wait()


def _tc_detile(tt):
    return pl.pallas_call(
        _detile_body,
        grid=(DT_GRID,),
        in_specs=[pl.BlockSpec((D, DT_W), lambda i: (0, i))],
        out_specs=pl.BlockSpec(memory_space=pl.ANY),
        out_shape=jax.ShapeDtypeStruct((VP, DP), jnp.float32),
        scratch_shapes=[
            pltpu.VMEM((2, DT_Q * DT_S, DP), jnp.float32),
            pltpu.SemaphoreType.DMA((2,)),
        ],
    )(tt)


def _fire(table_hbm, idx_v, rows_v, sem, c, b):
    for s in range(CHUNK // SUB):
        src = table_hbm.at[idx_v.at[pl.ds(c * CHUNK + s * SUB, SUB)]]
        pltpu.async_copy(src, rows_v.at[b, pl.ds(s * SUB, SUB)], sem.at[b])


def _drain(table_hbm, idx_v, rows_v, sem, b):
    for s in range(CHUNK // SUB):
        pltpu.make_async_copy(
            table_hbm.at[idx_v.at[pl.ds(s * SUB, SUB)]],
            rows_v.at[b, pl.ds(s * SUB, SUB)], sem.at[b]).wait()


def _sc_body(idx_hbm, table_hbm, out_hbm, idx_v, lb_v, rows_v, out_v, sem):
    wid = lax.axis_index("s") * NC + lax.axis_index("c")

    pltpu.sync_copy(idx_hbm.at[pl.ds(wid * IPW, IPW)], idx_v)

    def off_body(blk, carry):
        base = blk * PERIOD
        for v in range(PERIOD // L):
            off = ((lax.iota(jnp.int32, L) + v * L) % N_FIELDS) * FIELD_SIZE
            sl = pl.ds(base + v * L, L)
            r = idx_v[sl] + off
            idx_v[sl] = ((r >> 13) << 11) | (r & 2047)
            lb_v[sl] = ((r >> 11) & 3) << 5
        return carry

    lax.fori_loop(0, IPW // PERIOD, off_body, 0)

    for b in range(2):
        _fire(table_hbm, idx_v, rows_v, sem, b, b)

    def ring_body(i, carry):
        for b in range(2):
            c = 2 * i + b
            _drain(table_hbm, idx_v, rows_v, sem, b)
            lbase = c * CHUNK
            lbv = [lb_v[pl.ds(lbase + v * L, L)] for v in range(CHUNK // L)]
            for ii in range(GB):
                r0 = ii * N_FIELDS
                cb = lbv[r0 // L][r0 % L]
                acc0 = rows_v[b, r0, pl.ds(cb, L)]
                acc1 = rows_v[b, r0, pl.ds(cb + L, L)]
                for j in range(1, N_FIELDS):
                    k = r0 + j
                    cj = lbv[k // L][k % L]
                    acc0 = acc0 + rows_v[b, k, pl.ds(cj, L)]
                    acc1 = acc1 + rows_v[b, k, pl.ds(cj + L, L)]
                orow = c * GB + ii
                out_v[orow, pl.ds(0, L)] = acc0 * INV_N
                out_v[orow, pl.ds(L, L)] = acc1 * INV_N

            @pl.when(c + 2 < NCH)
            def _():
                _fire(table_hbm, idx_v, rows_v, sem, c + 2, b)
        return carry

    lax.fori_loop(0, NCH // 2, ring_body, 0)

    pltpu.sync_copy(out_v, out_hbm.at[pl.ds(wid * BPW, BPW)])


@jax.jit
def _sc_embed(idx_flat, table):
    mesh = plsc.VectorSubcoreMesh(core_axis_name="c", subcore_axis_name="s")
    return pl.kernel(
        _sc_body,
        out_type=jax.ShapeDtypeStruct((B, D), jnp.float32),
        mesh=mesh,
        scratch_types=[
            pltpu.VMEM((IPW,), jnp.int32),
            pltpu.VMEM((IPW,), jnp.int32),
            pltpu.VMEM((2, CHUNK, DP), jnp.float32),
            pltpu.VMEM((BPW, D), jnp.float32),
            pltpu.SemaphoreType.DMA((2,)),
        ],
        compiler_params=pltpu.CompilerParams(use_tc_tiling_on_sc=False),
    )(idx_flat, table)


def kernel(inputs, embedding):
    table_pack = _tc_detile(embedding.T)
    return _sc_embed(inputs.reshape(-1), table_pack)
